# baseline (speedup 1.0000x reference)
import functools

import jax
import jax.numpy as jnp
import numpy as np
from jax import lax
from jax.experimental import pallas as pl
from jax.experimental.pallas import tpu as pltpu
from jax.experimental.pallas import tpu_sc as plsc

N = 10000
E = 160000
WIDTH = 16
DEPTH = 4

NW = 32
EPW = E // NW
CH = 125
NCH = EPW // CH
RPS = N // 16


@functools.cache
def _sc_kernels():
    mesh = plsc.VectorSubcoreMesh(core_axis_name="c", subcore_axis_name="s")
    cparams = pltpu.CompilerParams(use_tc_tiling_on_sc=False)

    @functools.partial(
        pl.kernel,
        out_type=jax.ShapeDtypeStruct((E, WIDTH), jnp.float32),
        mesh=mesh,
        compiler_params=cparams,
        scratch_types=[
            pltpu.VMEM((NCH, CH), jnp.int32),
            pltpu.VMEM((EPW, WIDTH), jnp.float32),
            pltpu.VMEM_SHARED((N, WIDTH), jnp.float32),
            pltpu.SemaphoreType.DMA,
        ],
    )
    def sc_gather(h_hbm, src_hbm, out_hbm, idx_v, rows_v, h_sh, sem):
        c = lax.axis_index("c")
        s = lax.axis_index("s")
        wid = s * 2 + c

        @pl.when(s == 0)
        def _():
            pltpu.sync_copy(h_hbm, h_sh)

        pltpu.sync_copy(src_hbm.at[wid], idx_v)
        plsc.subcore_barrier()

        def fire(j, carry):
            pltpu.async_copy(h_sh.at[idx_v.at[j]],
                             rows_v.at[pl.ds(j * CH, CH)], sem)
            return carry

        lax.fori_loop(0, NCH, fire, 0)
        pltpu.make_async_copy(out_hbm.at[pl.ds(wid * EPW, EPW)], rows_v,
                              sem).wait()
        pltpu.sync_copy(rows_v, out_hbm.at[pl.ds(wid * EPW, EPW)])

    @functools.partial(
        pl.kernel,
        out_type=jax.ShapeDtypeStruct((2, N, WIDTH), jnp.float32),
        mesh=mesh,
        compiler_params=cparams,
        scratch_types=[
            pltpu.VMEM((NCH, CH), jnp.int32),
            pltpu.VMEM((EPW, WIDTH), jnp.float32),
            pltpu.VMEM_SHARED((N, WIDTH), jnp.float32),
        ],
    )
    def sc_scatter(msg_hbm, dst_hbm, zeros_hbm, out_hbm, idx_v, msg_v, agg_sh):
        c = lax.axis_index("c")
        s = lax.axis_index("s")
        wid = s * 2 + c

        @pl.when(s == 0)
        def _():
            pltpu.sync_copy(zeros_hbm, agg_sh)

        pltpu.sync_copy(dst_hbm.at[wid], idx_v)
        pltpu.sync_copy(msg_hbm.at[pl.ds(wid * EPW, EPW)], msg_v)
        plsc.subcore_barrier()

        def body(j, carry):
            pltpu.sync_copy(msg_v.at[pl.ds(j * CH, CH)],
                            agg_sh.at[idx_v.at[j]], add=True)
            return carry

        lax.fori_loop(0, NCH, body, 0)
        plsc.subcore_barrier()
        pltpu.sync_copy(agg_sh.at[pl.ds(s * RPS, RPS)],
                        out_hbm.at[c, pl.ds(s * RPS, RPS)])

    @functools.partial(
        pl.kernel,
        out_type=jax.ShapeDtypeStruct((2, N, WIDTH), jnp.float32),
        mesh=mesh,
        compiler_params=cparams,
        scratch_types=[
            pltpu.VMEM((NCH, CH), jnp.int32),
            pltpu.VMEM((CH, WIDTH), jnp.float32),
            pltpu.VMEM_SHARED((N, WIDTH), jnp.float32),
        ],
    )
    def sc_count(dst_hbm, zeros_hbm, ones_hbm, out_hbm, idx_v, ones_v, cnt_sh):
        c = lax.axis_index("c")
        s = lax.axis_index("s")
        wid = s * 2 + c

        @pl.when(s == 0)
        def _():
            pltpu.sync_copy(zeros_hbm, cnt_sh)

        pltpu.sync_copy(dst_hbm.at[wid], idx_v)
        pltpu.sync_copy(ones_hbm, ones_v)
        plsc.subcore_barrier()

        def body(j, carry):
            pltpu.sync_copy(ones_v, cnt_sh.at[idx_v.at[j]], add=True)
            return carry

        lax.fori_loop(0, NCH, body, 0)
        plsc.subcore_barrier()
        pltpu.sync_copy(cnt_sh.at[pl.ds(s * RPS, RPS)],
                        out_hbm.at[c, pl.ds(s * RPS, RPS)])

    return sc_gather, sc_scatter, sc_count


def _dot(a, b):
    return jax.lax.dot_general(a, b, (((1,), (0,)), ((), ())),
                               preferred_element_type=jnp.float32)


def _lift_body(x_ref, wf1, bf1, wf2, bf2, wc1, bc1, o_ref):
    x = x_ref[...]
    h = jnp.sin(_dot(x, wf1[...]) + bf1[...])
    o_ref[...] = _dot(h, wf2[...]) + bf2[...] + _dot(x, wc1[...]) + bc1[...]


def _bdot(a, b):
    return jax.lax.dot_general(a.astype(jnp.bfloat16), b.astype(jnp.bfloat16),
                               (((1,), (0,)), ((), ())),
                               preferred_element_type=jnp.float32)


def _edge_mlp_body(ea_ref, w1, b1, w2, b2, o_ref):
    e = jnp.maximum(_dot(ea_ref[...], w1[...]) + b1[...], 0.0)
    e = jnp.maximum(_dot(e, w2[...]) + b2[...], 0.0)
    o_ref[...] = e.astype(jnp.bfloat16)


def _msg_body(e3_ref, hs_ref, wk3, bk3, r2, s, o_ref):
    w = _bdot(e3_ref[...], wk3[...]) + bk3[...]
    hsx = _bdot(hs_ref[...], r2[...])
    o_ref[...] = _bdot(w * hsx, s[...])


def _inv_body(c0_ref, c1_ref, o_ref):
    o_ref[...] = 1.0 / jnp.maximum(c0_ref[...] + c1_ref[...], 1.0)


def _update_body(a0_ref, a1_ref, inv_ref, h_ref, root, cb, o_ref, *, relu):
    agg = (a0_ref[...] + a1_ref[...]) * inv_ref[...]
    h = agg + _dot(h_ref[...], root[...]) + cb[...]
    if relu:
        h = jnp.maximum(h, 0.0)
    o_ref[...] = h


def _final_body(a0_ref, a1_ref, inv_ref, h_ref, root, cb, wfc2, bfc2, o_ref):
    agg = (a0_ref[...] + a1_ref[...]) * inv_ref[...]
    h = agg + _dot(h_ref[...], root[...]) + cb[...]
    o_ref[...] = _dot(h, wfc2[...]) + bfc2[...]


def _full(x):
    return pl.BlockSpec(x.shape, lambda *_: tuple(0 for _ in x.shape))


def _expansion_mats():
    r2 = np.zeros((WIDTH, WIDTH * WIDTH), np.float32)
    s = np.zeros((WIDTH * WIDTH, WIDTH), np.float32)
    for i in range(WIDTH):
        for o in range(WIDTH):
            r2[i, WIDTH * i + o] = 1.0
            s[WIDTH * i + o, o] = 1.0
    return jnp.asarray(r2), jnp.asarray(s)


def kernel(x, edge_index, edge_attr, Wff1, bff1, Wff2, bff2, Wfc1, bfc1,
           Wk1, bk1, Wk2, bk2, Wk3, bk3, root, conv_bias, Wfc2, bfc2):
    f32 = jnp.float32
    sc_gather, sc_scatter, sc_count = _sc_kernels()

    src3 = edge_index[0].astype(jnp.int32).reshape(NW, NCH, CH)
    dst3 = edge_index[1].astype(jnp.int32).reshape(NW, NCH, CH)
    zeros_agg = jnp.zeros((N, WIDTH), f32)
    ones_ch = jnp.ones((CH, WIDTH), f32)
    r2, smat = _expansion_mats()

    cntp = sc_count(dst3, zeros_agg, ones_ch)
    inv16 = pl.pallas_call(
        _inv_body,
        out_shape=jax.ShapeDtypeStruct((N, WIDTH), f32),
    )(cntp[0], cntp[1])

    BN = 2000
    h = pl.pallas_call(
        _lift_body,
        grid=(N // BN,),
        in_specs=[pl.BlockSpec((BN, x.shape[1]), lambda i: (i, 0)),
                  _full(Wff1), _full(bff1.reshape(1, -1)),
                  _full(Wff2), _full(bff2.reshape(1, -1)),
                  _full(Wfc1), _full(bfc1.reshape(1, -1))],
        out_specs=pl.BlockSpec((BN, WIDTH), lambda i: (i, 0)),
        out_shape=jax.ShapeDtypeStruct((N, WIDTH), f32),
    )(x, Wff1, bff1.reshape(1, -1), Wff2, bff2.reshape(1, -1),
      Wfc1, bfc1.reshape(1, -1))

    b1_2d = bk1.reshape(1, -1)
    b2_2d = bk2.reshape(1, -1)
    bk3_2d = bk3.reshape(1, -1)

    BE0 = 4000
    KDIM = Wk2.shape[1]
    e3 = pl.pallas_call(
        _edge_mlp_body,
        grid=(E // BE0,),
        in_specs=[pl.BlockSpec((BE0, edge_attr.shape[1]), lambda i: (i, 0)),
                  _full(Wk1), _full(b1_2d), _full(Wk2), _full(b2_2d)],
        out_specs=pl.BlockSpec((BE0, KDIM), lambda i: (i, 0)),
        out_shape=jax.ShapeDtypeStruct((E, KDIM), jnp.bfloat16),
    )(edge_attr, Wk1, b1_2d, Wk2, b2_2d)
    cb_2d = conv_bias.reshape(1, -1)
    bfc2_2d = bfc2.reshape(1, -1)

    BE = 4000
    for k in range(DEPTH):
        hs = sc_gather(h, src3)
        msg = pl.pallas_call(
            _msg_body,
            grid=(E // BE,),
            in_specs=[pl.BlockSpec((BE, KDIM), lambda i: (i, 0)),
                      pl.BlockSpec((BE, WIDTH), lambda i: (i, 0)),
                      _full(Wk3), _full(bk3_2d), _full(r2), _full(smat)],
            out_specs=pl.BlockSpec((BE, WIDTH), lambda i: (i, 0)),
            out_shape=jax.ShapeDtypeStruct((E, WIDTH), f32),
        )(e3, hs, Wk3, bk3_2d, r2, smat)
        aggp = sc_scatter(msg, dst3, zeros_agg)
        if k != DEPTH - 1:
            h = pl.pallas_call(
                functools.partial(_update_body, relu=True),
                out_shape=jax.ShapeDtypeStruct((N, WIDTH), f32),
            )(aggp[0], aggp[1], inv16, h, root, cb_2d)
        else:
            out = pl.pallas_call(
                _final_body,
                out_shape=jax.ShapeDtypeStruct((N, 1), f32),
            )(aggp[0], aggp[1], inv16, h, root, cb_2d, Wfc2, bfc2_2d)
    return out


# speedup vs baseline: 1.5277x; 1.5277x over previous
import functools

import jax
import jax.numpy as jnp
import numpy as np
from jax import lax
from jax.experimental import pallas as pl
from jax.experimental.pallas import tpu as pltpu
from jax.experimental.pallas import tpu_sc as plsc

N = 10000
E = 160000
WIDTH = 16
DEPTH = 4

NW = 32
EPW = E // NW
CH = 125
NCH = EPW // CH
RPS = N // 16


@functools.cache
def _sc_kernels():
    mesh = plsc.VectorSubcoreMesh(core_axis_name="c", subcore_axis_name="s")
    cparams = pltpu.CompilerParams(use_tc_tiling_on_sc=False)

    @functools.partial(
        pl.kernel,
        out_type=jax.ShapeDtypeStruct((E, WIDTH), jnp.float32),
        mesh=mesh,
        compiler_params=cparams,
        scratch_types=[
            pltpu.VMEM((NCH, CH), jnp.int32),
            pltpu.VMEM((EPW, WIDTH), jnp.float32),
            pltpu.VMEM_SHARED((N, WIDTH), jnp.float32),
            pltpu.SemaphoreType.DMA,
        ],
    )
    def sc_gather(h_hbm, src_hbm, out_hbm, idx_v, rows_v, h_sh, sem):
        c = lax.axis_index("c")
        s = lax.axis_index("s")
        wid = s * 2 + c

        @pl.when(s == 0)
        def _():
            pltpu.sync_copy(h_hbm, h_sh)

        pltpu.sync_copy(src_hbm.at[wid], idx_v)
        plsc.subcore_barrier()

        def fire(j, carry):
            pltpu.async_copy(h_sh.at[idx_v.at[j]],
                             rows_v.at[pl.ds(j * CH, CH)], sem)
            return carry

        lax.fori_loop(0, NCH, fire, 0)
        pltpu.make_async_copy(out_hbm.at[pl.ds(wid * EPW, EPW)], rows_v,
                              sem).wait()
        pltpu.sync_copy(rows_v, out_hbm.at[pl.ds(wid * EPW, EPW)])

    @functools.partial(
        pl.kernel,
        out_type=jax.ShapeDtypeStruct((2, N, WIDTH), jnp.float32),
        mesh=mesh,
        compiler_params=cparams,
        scratch_types=[
            pltpu.VMEM((NCH, CH), jnp.int32),
            pltpu.VMEM((EPW, WIDTH), jnp.float32),
            pltpu.VMEM_SHARED((N, WIDTH), jnp.float32),
        ],
    )
    def sc_scatter(msg_hbm, dst_hbm, zeros_hbm, out_hbm, idx_v, msg_v, agg_sh):
        c = lax.axis_index("c")
        s = lax.axis_index("s")
        wid = s * 2 + c

        @pl.when(s == 0)
        def _():
            pltpu.sync_copy(zeros_hbm, agg_sh)

        pltpu.sync_copy(dst_hbm.at[wid], idx_v)
        pltpu.sync_copy(msg_hbm.at[pl.ds(wid * EPW, EPW)], msg_v)
        plsc.subcore_barrier()

        def body(j, carry):
            pltpu.sync_copy(msg_v.at[pl.ds(j * CH, CH)],
                            agg_sh.at[idx_v.at[j]], add=True)
            return carry

        lax.fori_loop(0, NCH, body, 0)
        plsc.subcore_barrier()
        pltpu.sync_copy(agg_sh.at[pl.ds(s * RPS, RPS)],
                        out_hbm.at[c, pl.ds(s * RPS, RPS)])

    @functools.partial(
        pl.kernel,
        out_type=jax.ShapeDtypeStruct((2, N, WIDTH), jnp.float32),
        mesh=mesh,
        compiler_params=cparams,
        scratch_types=[
            pltpu.VMEM((NCH, CH), jnp.int32),
            pltpu.VMEM((CH, WIDTH), jnp.float32),
            pltpu.VMEM_SHARED((N, WIDTH), jnp.float32),
        ],
    )
    def sc_count(dst_hbm, zeros_hbm, ones_hbm, out_hbm, idx_v, ones_v, cnt_sh):
        c = lax.axis_index("c")
        s = lax.axis_index("s")
        wid = s * 2 + c

        @pl.when(s == 0)
        def _():
            pltpu.sync_copy(zeros_hbm, cnt_sh)

        pltpu.sync_copy(dst_hbm.at[wid], idx_v)
        pltpu.sync_copy(ones_hbm, ones_v)
        plsc.subcore_barrier()

        def body(j, carry):
            pltpu.sync_copy(ones_v, cnt_sh.at[idx_v.at[j]], add=True)
            return carry

        lax.fori_loop(0, NCH, body, 0)
        plsc.subcore_barrier()
        pltpu.sync_copy(cnt_sh.at[pl.ds(s * RPS, RPS)],
                        out_hbm.at[c, pl.ds(s * RPS, RPS)])

    return sc_gather, sc_scatter, sc_count


def _dot(a, b):
    return jax.lax.dot_general(a, b, (((1,), (0,)), ((), ())),
                               preferred_element_type=jnp.float32)


def _lift_body(x_ref, wf1, bf1, wf2, bf2, wc1, bc1, o_ref):
    x = x_ref[...]
    h = jnp.sin(_dot(x, wf1[...]) + bf1[...])
    o_ref[...] = _dot(h, wf2[...]) + bf2[...] + _dot(x, wc1[...]) + bc1[...]


def _bdot(a, b):
    return jax.lax.dot_general(a.astype(jnp.bfloat16), b.astype(jnp.bfloat16),
                               (((1,), (0,)), ((), ())),
                               preferred_element_type=jnp.float32)


def _msg_body(ea_ref, hsp_ref, w1, b1, w2, b2, wk3, bk3, r2, s, o_ref):
    e = jnp.maximum(_dot(ea_ref[...], w1[...]) + b1[...], 0.0)
    e = jnp.maximum(_dot(e, w2[...]) + b2[...], 0.0)
    w = _bdot(e, wk3[...]) + bk3[...]
    hsp = hsp_ref[...]
    br = hsp.shape[0]
    cols = []
    for a in range(8):
        hsx_a = _bdot(hsp[:, 16 * a:16 * (a + 1)], r2[...])
        t_a = w[a * br:(a + 1) * br, :] * hsx_a
        cols.append(_bdot(t_a, s[...]))
    o_ref[...] = jnp.concatenate(cols, axis=1)


def _inv_body(c0_ref, c1_ref, o_ref):
    o_ref[...] = 1.0 / jnp.maximum(c0_ref[...] + c1_ref[...], 1.0)


def _update_body(a0_ref, a1_ref, inv_ref, h_ref, root, cb, o_ref, *, relu):
    agg = (a0_ref[...] + a1_ref[...]) * inv_ref[...]
    h = agg + _dot(h_ref[...], root[...]) + cb[...]
    if relu:
        h = jnp.maximum(h, 0.0)
    o_ref[...] = h


def _final_body(a0_ref, a1_ref, inv_ref, h_ref, root, cb, wfc2, bfc2, o_ref):
    agg = (a0_ref[...] + a1_ref[...]) * inv_ref[...]
    h = agg + _dot(h_ref[...], root[...]) + cb[...]
    o_ref[...] = _dot(h, wfc2[...]) + bfc2[...]


def _full(x):
    return pl.BlockSpec(x.shape, lambda *_: tuple(0 for _ in x.shape))


def _expansion_mats():
    r2 = np.zeros((WIDTH, WIDTH * WIDTH), np.float32)
    s = np.zeros((WIDTH * WIDTH, WIDTH), np.float32)
    for i in range(WIDTH):
        for o in range(WIDTH):
            r2[i, WIDTH * i + o] = 1.0
            s[WIDTH * i + o, o] = 1.0
    return jnp.asarray(r2), jnp.asarray(s)


def kernel(x, edge_index, edge_attr, Wff1, bff1, Wff2, bff2, Wfc1, bfc1,
           Wk1, bk1, Wk2, bk2, Wk3, bk3, root, conv_bias, Wfc2, bfc2):
    f32 = jnp.float32
    sc_gather, sc_scatter, sc_count = _sc_kernels()

    src3 = edge_index[0].astype(jnp.int32).reshape(NW, NCH, CH)
    dst3 = edge_index[1].astype(jnp.int32).reshape(NW, NCH, CH)
    zeros_agg = jnp.zeros((N, WIDTH), f32)
    ones_ch = jnp.ones((CH, WIDTH), f32)
    r2, smat = _expansion_mats()

    cntp = sc_count(dst3, zeros_agg, ones_ch)
    inv16 = pl.pallas_call(
        _inv_body,
        out_shape=jax.ShapeDtypeStruct((N, WIDTH), f32),
    )(cntp[0], cntp[1])

    BN = 2000
    h = pl.pallas_call(
        _lift_body,
        grid=(N // BN,),
        in_specs=[pl.BlockSpec((BN, x.shape[1]), lambda i: (i, 0)),
                  _full(Wff1), _full(bff1.reshape(1, -1)),
                  _full(Wff2), _full(bff2.reshape(1, -1)),
                  _full(Wfc1), _full(bfc1.reshape(1, -1))],
        out_specs=pl.BlockSpec((BN, WIDTH), lambda i: (i, 0)),
        out_shape=jax.ShapeDtypeStruct((N, WIDTH), f32),
    )(x, Wff1, bff1.reshape(1, -1), Wff2, bff2.reshape(1, -1),
      Wfc1, bfc1.reshape(1, -1))

    b1_2d = bk1.reshape(1, -1)
    b2_2d = bk2.reshape(1, -1)
    bk3_2d = bk3.reshape(1, -1)
    cb_2d = conv_bias.reshape(1, -1)
    bfc2_2d = bfc2.reshape(1, -1)

    BE = 3200
    BR = BE // 8
    perm = np.arange(E).reshape(E // BE, BR, 8).transpose(0, 2, 1).reshape(E)
    ea_perm = edge_attr[jnp.asarray(perm)]
    for k in range(DEPTH):
        hs = sc_gather(h, src3)
        hs_p = hs.reshape(E // 8, 128)
        msg_p = pl.pallas_call(
            _msg_body,
            grid=(E // BE,),
            in_specs=[pl.BlockSpec((BE, ea_perm.shape[1]), lambda i: (i, 0)),
                      pl.BlockSpec((BR, 128), lambda i: (i, 0)),
                      _full(Wk1), _full(b1_2d), _full(Wk2), _full(b2_2d),
                      _full(Wk3), _full(bk3_2d), _full(r2), _full(smat)],
            out_specs=pl.BlockSpec((BR, 128), lambda i: (i, 0)),
            out_shape=jax.ShapeDtypeStruct((E // 8, 128), f32),
        )(ea_perm, hs_p, Wk1, b1_2d, Wk2, b2_2d, Wk3, bk3_2d, r2, smat)
        aggp = sc_scatter(msg_p.reshape(E, WIDTH), dst3, zeros_agg)
        if k != DEPTH - 1:
            h = pl.pallas_call(
                functools.partial(_update_body, relu=True),
                out_shape=jax.ShapeDtypeStruct((N, WIDTH), f32),
            )(aggp[0], aggp[1], inv16, h, root, cb_2d)
        else:
            out = pl.pallas_call(
                _final_body,
                out_shape=jax.ShapeDtypeStruct((N, 1), f32),
            )(aggp[0], aggp[1], inv16, h, root, cb_2d, Wfc2, bfc2_2d)
    return out


# speedup vs baseline: 1.5286x; 1.0006x over previous
import functools

import jax
import jax.numpy as jnp
import numpy as np
from jax import lax
from jax.experimental import pallas as pl
from jax.experimental.pallas import tpu as pltpu
from jax.experimental.pallas import tpu_sc as plsc

N = 10000
E = 160000
WIDTH = 16
DEPTH = 4

NW = 32
EPW = E // NW
CH = 125
NCH = EPW // CH
RPS = N // 16


@functools.cache
def _sc_kernels():
    mesh = plsc.VectorSubcoreMesh(core_axis_name="c", subcore_axis_name="s")
    cparams = pltpu.CompilerParams(use_tc_tiling_on_sc=False)

    @functools.partial(
        pl.kernel,
        out_type=jax.ShapeDtypeStruct((E, WIDTH), jnp.float32),
        mesh=mesh,
        compiler_params=cparams,
        scratch_types=[
            pltpu.VMEM((NCH, CH), jnp.int32),
            pltpu.VMEM((EPW, WIDTH), jnp.float32),
            pltpu.VMEM_SHARED((N, WIDTH), jnp.float32),
            pltpu.SemaphoreType.DMA,
        ],
    )
    def sc_gather(h_hbm, src_hbm, out_hbm, idx_v, rows_v, h_sh, sem):
        c = lax.axis_index("c")
        s = lax.axis_index("s")
        wid = s * 2 + c

        @pl.when(s == 0)
        def _():
            pltpu.sync_copy(h_hbm, h_sh)

        pltpu.sync_copy(src_hbm.at[wid], idx_v)
        plsc.subcore_barrier()

        def fire(j, carry):
            pltpu.async_copy(h_sh.at[idx_v.at[j]],
                             rows_v.at[pl.ds(j * CH, CH)], sem)
            return carry

        lax.fori_loop(0, NCH, fire, 0)
        pltpu.make_async_copy(out_hbm.at[pl.ds(wid * EPW, EPW)], rows_v,
                              sem).wait()
        pltpu.sync_copy(rows_v, out_hbm.at[pl.ds(wid * EPW, EPW)])

    @functools.partial(
        pl.kernel,
        out_type=jax.ShapeDtypeStruct((2, N, WIDTH), jnp.float32),
        mesh=mesh,
        compiler_params=cparams,
        scratch_types=[
            pltpu.VMEM((NCH, CH), jnp.int32),
            pltpu.VMEM((EPW, WIDTH), jnp.float32),
            pltpu.VMEM_SHARED((N, WIDTH), jnp.float32),
        ],
    )
    def sc_scatter(msg_hbm, dst_hbm, zeros_hbm, out_hbm, idx_v, msg_v, agg_sh):
        c = lax.axis_index("c")
        s = lax.axis_index("s")
        wid = s * 2 + c

        @pl.when(s == 0)
        def _():
            pltpu.sync_copy(zeros_hbm, agg_sh)

        pltpu.sync_copy(dst_hbm.at[wid], idx_v)
        pltpu.sync_copy(msg_hbm.at[pl.ds(wid * EPW, EPW)], msg_v)
        plsc.subcore_barrier()

        def body(j, carry):
            pltpu.sync_copy(msg_v.at[pl.ds(j * CH, CH)],
                            agg_sh.at[idx_v.at[j]], add=True)
            return carry

        lax.fori_loop(0, NCH, body, 0)
        plsc.subcore_barrier()
        pltpu.sync_copy(agg_sh.at[pl.ds(s * RPS, RPS)],
                        out_hbm.at[c, pl.ds(s * RPS, RPS)])

    @functools.partial(
        pl.kernel,
        out_type=jax.ShapeDtypeStruct((2, N, WIDTH), jnp.float32),
        mesh=mesh,
        compiler_params=cparams,
        scratch_types=[
            pltpu.VMEM((NCH, CH), jnp.int32),
            pltpu.VMEM((CH, WIDTH), jnp.float32),
            pltpu.VMEM_SHARED((N, WIDTH), jnp.float32),
        ],
    )
    def sc_count(dst_hbm, zeros_hbm, ones_hbm, out_hbm, idx_v, ones_v, cnt_sh):
        c = lax.axis_index("c")
        s = lax.axis_index("s")
        wid = s * 2 + c

        @pl.when(s == 0)
        def _():
            pltpu.sync_copy(zeros_hbm, cnt_sh)

        pltpu.sync_copy(dst_hbm.at[wid], idx_v)
        pltpu.sync_copy(ones_hbm, ones_v)
        plsc.subcore_barrier()

        def body(j, carry):
            pltpu.sync_copy(ones_v, cnt_sh.at[idx_v.at[j]], add=True)
            return carry

        lax.fori_loop(0, NCH, body, 0)
        plsc.subcore_barrier()
        pltpu.sync_copy(cnt_sh.at[pl.ds(s * RPS, RPS)],
                        out_hbm.at[c, pl.ds(s * RPS, RPS)])

    return sc_gather, sc_scatter, sc_count


def _dot(a, b):
    return jax.lax.dot_general(a, b, (((1,), (0,)), ((), ())),
                               preferred_element_type=jnp.float32)


def _lift_body(x_ref, wf1, bf1, wf2, bf2, wc1, bc1, o_ref):
    x = x_ref[...]
    h = jnp.sin(_dot(x, wf1[...]) + bf1[...])
    o_ref[...] = _dot(h, wf2[...]) + bf2[...] + _dot(x, wc1[...]) + bc1[...]


def _bdot(a, b, out=jnp.float32):
    return jax.lax.dot_general(a.astype(jnp.bfloat16), b.astype(jnp.bfloat16),
                               (((1,), (0,)), ((), ())),
                               preferred_element_type=out)


def _msg_body(ea_ref, hsp_ref, w1, b1, w2, b2, wk3, bk3, r2, s, o_ref):
    e = jnp.maximum(_dot(ea_ref[...], w1[...]) + b1[...], 0.0)
    e = jnp.maximum(_dot(e, w2[...]) + b2[...], 0.0)
    w = _bdot(e, wk3[...]) + bk3[...]
    hsp = hsp_ref[...]
    br = hsp.shape[0]
    cols = []
    for a in range(8):
        hsx_a = _bdot(hsp[:, 16 * a:16 * (a + 1)], r2[...])
        t_a = w[a * br:(a + 1) * br, :] * hsx_a
        cols.append(_bdot(t_a, s[...]))
    o_ref[...] = jnp.concatenate(cols, axis=1)


def _inv_body(c0_ref, c1_ref, o_ref):
    o_ref[...] = 1.0 / jnp.maximum(c0_ref[...] + c1_ref[...], 1.0)


def _update_body(a0_ref, a1_ref, inv_ref, h_ref, root, cb, o_ref, *, relu):
    agg = (a0_ref[...] + a1_ref[...]) * inv_ref[...]
    h = agg + _dot(h_ref[...], root[...]) + cb[...]
    if relu:
        h = jnp.maximum(h, 0.0)
    o_ref[...] = h


def _final_body(a0_ref, a1_ref, inv_ref, h_ref, root, cb, wfc2, bfc2, o_ref):
    agg = (a0_ref[...] + a1_ref[...]) * inv_ref[...]
    h = agg + _dot(h_ref[...], root[...]) + cb[...]
    o_ref[...] = _dot(h, wfc2[...]) + bfc2[...]


def _full(x):
    return pl.BlockSpec(x.shape, lambda *_: tuple(0 for _ in x.shape))


def _expansion_mats():
    r2 = np.zeros((WIDTH, WIDTH * WIDTH), np.float32)
    s = np.zeros((WIDTH * WIDTH, WIDTH), np.float32)
    for i in range(WIDTH):
        for o in range(WIDTH):
            r2[i, WIDTH * i + o] = 1.0
            s[WIDTH * i + o, o] = 1.0
    return jnp.asarray(r2), jnp.asarray(s)


def kernel(x, edge_index, edge_attr, Wff1, bff1, Wff2, bff2, Wfc1, bfc1,
           Wk1, bk1, Wk2, bk2, Wk3, bk3, root, conv_bias, Wfc2, bfc2):
    f32 = jnp.float32
    sc_gather, sc_scatter, sc_count = _sc_kernels()

    src3 = edge_index[0].astype(jnp.int32).reshape(NW, NCH, CH)
    dst3 = edge_index[1].astype(jnp.int32).reshape(NW, NCH, CH)
    zeros_agg = jnp.zeros((N, WIDTH), f32)
    ones_ch = jnp.ones((CH, WIDTH), f32)
    r2, smat = _expansion_mats()

    cntp = sc_count(dst3, zeros_agg, ones_ch)
    inv16 = pl.pallas_call(
        _inv_body,
        out_shape=jax.ShapeDtypeStruct((N, WIDTH), f32),
    )(cntp[0], cntp[1])

    BN = 2000
    h = pl.pallas_call(
        _lift_body,
        grid=(N // BN,),
        in_specs=[pl.BlockSpec((BN, x.shape[1]), lambda i: (i, 0)),
                  _full(Wff1), _full(bff1.reshape(1, -1)),
                  _full(Wff2), _full(bff2.reshape(1, -1)),
                  _full(Wfc1), _full(bfc1.reshape(1, -1))],
        out_specs=pl.BlockSpec((BN, WIDTH), lambda i: (i, 0)),
        out_shape=jax.ShapeDtypeStruct((N, WIDTH), f32),
    )(x, Wff1, bff1.reshape(1, -1), Wff2, bff2.reshape(1, -1),
      Wfc1, bfc1.reshape(1, -1))

    b1_2d = bk1.reshape(1, -1)
    b2_2d = bk2.reshape(1, -1)
    bk3_2d = bk3.reshape(1, -1)
    cb_2d = conv_bias.reshape(1, -1)
    bfc2_2d = bfc2.reshape(1, -1)

    BE = 3200
    BR = BE // 8
    perm = np.arange(E).reshape(E // BE, BR, 8).transpose(0, 2, 1).reshape(E)
    ea_perm = edge_attr[jnp.asarray(perm)]
    for k in range(DEPTH):
        hs = sc_gather(h, src3)
        hs_p = hs.reshape(E // 8, 128)
        msg_p = pl.pallas_call(
            _msg_body,
            grid=(E // BE,),
            in_specs=[pl.BlockSpec((BE, ea_perm.shape[1]), lambda i: (i, 0)),
                      pl.BlockSpec((BR, 128), lambda i: (i, 0)),
                      _full(Wk1), _full(b1_2d), _full(Wk2), _full(b2_2d),
                      _full(Wk3), _full(bk3_2d), _full(r2), _full(smat)],
            out_specs=pl.BlockSpec((BR, 128), lambda i: (i, 0)),
            out_shape=jax.ShapeDtypeStruct((E // 8, 128), f32),
        )(ea_perm, hs_p, Wk1, b1_2d, Wk2, b2_2d, Wk3, bk3_2d, r2, smat)
        aggp = sc_scatter(msg_p.reshape(E, WIDTH), dst3, zeros_agg)
        if k != DEPTH - 1:
            h = pl.pallas_call(
                functools.partial(_update_body, relu=True),
                out_shape=jax.ShapeDtypeStruct((N, WIDTH), f32),
            )(aggp[0], aggp[1], inv16, h, root, cb_2d)
        else:
            out = pl.pallas_call(
                _final_body,
                out_shape=jax.ShapeDtypeStruct((N, 1), f32),
            )(aggp[0], aggp[1], inv16, h, root, cb_2d, Wfc2, bfc2_2d)
    return out


# speedup vs baseline: 1.5785x; 1.0326x over previous
import functools

import jax
import jax.numpy as jnp
import numpy as np
from jax import lax
from jax.experimental import pallas as pl
from jax.experimental.pallas import tpu as pltpu
from jax.experimental.pallas import tpu_sc as plsc

N = 10000
E = 160000
WIDTH = 16
DEPTH = 4

NW = 32
EPW = E // NW
CH = 125
NCH = EPW // CH
RPS = N // 16


@functools.cache
def _sc_kernels():
    mesh = plsc.VectorSubcoreMesh(core_axis_name="c", subcore_axis_name="s")
    cparams = pltpu.CompilerParams(use_tc_tiling_on_sc=False)

    @functools.partial(
        pl.kernel,
        out_type=jax.ShapeDtypeStruct((E, WIDTH), jnp.float32),
        mesh=mesh,
        compiler_params=cparams,
        scratch_types=[
            pltpu.VMEM((NCH, CH), jnp.int32),
            pltpu.VMEM((EPW, WIDTH), jnp.float32),
            pltpu.VMEM_SHARED((N, WIDTH), jnp.float32),
            pltpu.SemaphoreType.DMA,
        ],
    )
    def sc_gather(h_hbm, src_hbm, out_hbm, idx_v, rows_v, h_sh, sem):
        c = lax.axis_index("c")
        s = lax.axis_index("s")
        wid = s * 2 + c

        @pl.when(s == 0)
        def _():
            pltpu.sync_copy(h_hbm, h_sh)

        pltpu.sync_copy(src_hbm.at[wid], idx_v)
        plsc.subcore_barrier()

        def fire(j, carry):
            pltpu.async_copy(h_sh.at[idx_v.at[j]],
                             rows_v.at[pl.ds(j * CH, CH)], sem)
            return carry

        lax.fori_loop(0, NCH, fire, 0)
        pltpu.make_async_copy(out_hbm.at[pl.ds(wid * EPW, EPW)], rows_v,
                              sem).wait()
        pltpu.sync_copy(rows_v, out_hbm.at[pl.ds(wid * EPW, EPW)])

    @functools.partial(
        pl.kernel,
        out_type=jax.ShapeDtypeStruct((2, N, WIDTH), jnp.float32),
        mesh=mesh,
        compiler_params=cparams,
        scratch_types=[
            pltpu.VMEM((NCH, CH), jnp.int32),
            pltpu.VMEM((EPW, WIDTH), jnp.float32),
            pltpu.VMEM_SHARED((N, WIDTH), jnp.float32),
        ],
    )
    def sc_scatter(msg_hbm, dst_hbm, zeros_hbm, out_hbm, idx_v, msg_v, agg_sh):
        c = lax.axis_index("c")
        s = lax.axis_index("s")
        wid = s * 2 + c

        @pl.when(s == 0)
        def _():
            pltpu.sync_copy(zeros_hbm, agg_sh)

        pltpu.sync_copy(dst_hbm.at[wid], idx_v)
        pltpu.sync_copy(msg_hbm.at[pl.ds(wid * EPW, EPW)], msg_v)
        plsc.subcore_barrier()

        def body(j, carry):
            pltpu.sync_copy(msg_v.at[pl.ds(j * CH, CH)],
                            agg_sh.at[idx_v.at[j]], add=True)
            return carry

        lax.fori_loop(0, NCH, body, 0)
        plsc.subcore_barrier()
        pltpu.sync_copy(agg_sh.at[pl.ds(s * RPS, RPS)],
                        out_hbm.at[c, pl.ds(s * RPS, RPS)])

    @functools.partial(
        pl.kernel,
        out_type=jax.ShapeDtypeStruct((2, N, WIDTH), jnp.float32),
        mesh=mesh,
        compiler_params=cparams,
        scratch_types=[
            pltpu.VMEM((NCH, CH), jnp.int32),
            pltpu.VMEM((CH, WIDTH), jnp.float32),
            pltpu.VMEM_SHARED((N, WIDTH), jnp.float32),
        ],
    )
    def sc_count(dst_hbm, zeros_hbm, ones_hbm, out_hbm, idx_v, ones_v, cnt_sh):
        c = lax.axis_index("c")
        s = lax.axis_index("s")
        wid = s * 2 + c

        @pl.when(s == 0)
        def _():
            pltpu.sync_copy(zeros_hbm, cnt_sh)

        pltpu.sync_copy(dst_hbm.at[wid], idx_v)
        pltpu.sync_copy(ones_hbm, ones_v)
        plsc.subcore_barrier()

        def body(j, carry):
            pltpu.sync_copy(ones_v, cnt_sh.at[idx_v.at[j]], add=True)
            return carry

        lax.fori_loop(0, NCH, body, 0)
        plsc.subcore_barrier()
        pltpu.sync_copy(cnt_sh.at[pl.ds(s * RPS, RPS)],
                        out_hbm.at[c, pl.ds(s * RPS, RPS)])

    return sc_gather, sc_scatter, sc_count


def _dot(a, b):
    return jax.lax.dot_general(a, b, (((1,), (0,)), ((), ())),
                               preferred_element_type=jnp.float32)


def _lift_body(x_ref, wf1, bf1, wf2, bf2, wc1, bc1, o_ref):
    x = x_ref[...]
    h = jnp.sin(_dot(x, wf1[...]) + bf1[...])
    o_ref[...] = _dot(h, wf2[...]) + bf2[...] + _dot(x, wc1[...]) + bc1[...]


def _bdot(a, b, out=jnp.float32):
    return jax.lax.dot_general(a.astype(jnp.bfloat16), b.astype(jnp.bfloat16),
                               (((1,), (0,)), ((), ())),
                               preferred_element_type=out)


def _edge_mlp_body(ea_ref, w1, b1, w2, b2, o_ref):
    e = jnp.maximum(_dot(ea_ref[...], w1[...]) + b1[...], 0.0)
    e = jnp.maximum(_dot(e, w2[...]) + b2[...], 0.0)
    o_ref[...] = e.astype(jnp.bfloat16)


def _msg_body(e3_ref, hsp_ref, wk3, bk3, r2, s, o_ref):
    w = _bdot(e3_ref[...], wk3[...]) + bk3[...]
    hsp = hsp_ref[...]
    br = hsp.shape[0]
    cols = []
    for a in range(8):
        hsx_a = _bdot(hsp[:, 16 * a:16 * (a + 1)], r2[...])
        t_a = w[a * br:(a + 1) * br, :] * hsx_a
        cols.append(_bdot(t_a, s[...]))
    o_ref[...] = jnp.concatenate(cols, axis=1)


def _inv_body(c0_ref, c1_ref, o_ref):
    o_ref[...] = 1.0 / jnp.maximum(c0_ref[...] + c1_ref[...], 1.0)


def _update_body(a0_ref, a1_ref, inv_ref, h_ref, root, cb, o_ref, *, relu):
    agg = (a0_ref[...] + a1_ref[...]) * inv_ref[...]
    h = agg + _dot(h_ref[...], root[...]) + cb[...]
    if relu:
        h = jnp.maximum(h, 0.0)
    o_ref[...] = h


def _final_body(a0_ref, a1_ref, inv_ref, h_ref, root, cb, wfc2, bfc2, o_ref):
    agg = (a0_ref[...] + a1_ref[...]) * inv_ref[...]
    h = agg + _dot(h_ref[...], root[...]) + cb[...]
    o_ref[...] = _dot(h, wfc2[...]) + bfc2[...]


def _full(x):
    return pl.BlockSpec(x.shape, lambda *_: tuple(0 for _ in x.shape))


def _expansion_mats():
    r2 = np.zeros((WIDTH, WIDTH * WIDTH), np.float32)
    s = np.zeros((WIDTH * WIDTH, WIDTH), np.float32)
    for i in range(WIDTH):
        for o in range(WIDTH):
            r2[i, WIDTH * i + o] = 1.0
            s[WIDTH * i + o, o] = 1.0
    return jnp.asarray(r2), jnp.asarray(s)


def kernel(x, edge_index, edge_attr, Wff1, bff1, Wff2, bff2, Wfc1, bfc1,
           Wk1, bk1, Wk2, bk2, Wk3, bk3, root, conv_bias, Wfc2, bfc2):
    f32 = jnp.float32
    sc_gather, sc_scatter, sc_count = _sc_kernels()

    src3 = edge_index[0].astype(jnp.int32).reshape(NW, NCH, CH)
    dst3 = edge_index[1].astype(jnp.int32).reshape(NW, NCH, CH)
    zeros_agg = jnp.zeros((N, WIDTH), f32)
    ones_ch = jnp.ones((CH, WIDTH), f32)
    r2, smat = _expansion_mats()

    cntp = sc_count(dst3, zeros_agg, ones_ch)
    inv16 = pl.pallas_call(
        _inv_body,
        out_shape=jax.ShapeDtypeStruct((N, WIDTH), f32),
    )(cntp[0], cntp[1])

    BN = 2000
    h = pl.pallas_call(
        _lift_body,
        grid=(N // BN,),
        in_specs=[pl.BlockSpec((BN, x.shape[1]), lambda i: (i, 0)),
                  _full(Wff1), _full(bff1.reshape(1, -1)),
                  _full(Wff2), _full(bff2.reshape(1, -1)),
                  _full(Wfc1), _full(bfc1.reshape(1, -1))],
        out_specs=pl.BlockSpec((BN, WIDTH), lambda i: (i, 0)),
        out_shape=jax.ShapeDtypeStruct((N, WIDTH), f32),
    )(x, Wff1, bff1.reshape(1, -1), Wff2, bff2.reshape(1, -1),
      Wfc1, bfc1.reshape(1, -1))

    b1_2d = bk1.reshape(1, -1)
    b2_2d = bk2.reshape(1, -1)
    bk3_2d = bk3.reshape(1, -1)
    cb_2d = conv_bias.reshape(1, -1)
    bfc2_2d = bfc2.reshape(1, -1)

    BE = 3200
    BR = BE // 8
    perm = np.arange(E).reshape(E // BE, BR, 8).transpose(0, 2, 1).reshape(E)
    ea_perm = edge_attr[jnp.asarray(perm)]
    KDIM = Wk2.shape[1]
    e3p = pl.pallas_call(
        _edge_mlp_body,
        grid=(E // BE,),
        in_specs=[pl.BlockSpec((BE, ea_perm.shape[1]), lambda i: (i, 0)),
                  _full(Wk1), _full(b1_2d), _full(Wk2), _full(b2_2d)],
        out_specs=pl.BlockSpec((BE, KDIM), lambda i: (i, 0)),
        out_shape=jax.ShapeDtypeStruct((E, KDIM), jnp.bfloat16),
    )(ea_perm, Wk1, b1_2d, Wk2, b2_2d)
    for k in range(DEPTH):
        hs = sc_gather(h, src3)
        hs_p = hs.reshape(E // 8, 128)
        msg_p = pl.pallas_call(
            _msg_body,
            grid=(E // BE,),
            in_specs=[pl.BlockSpec((BE, KDIM), lambda i: (i, 0)),
                      pl.BlockSpec((BR, 128), lambda i: (i, 0)),
                      _full(Wk3), _full(bk3_2d), _full(r2), _full(smat)],
            out_specs=pl.BlockSpec((BR, 128), lambda i: (i, 0)),
            out_shape=jax.ShapeDtypeStruct((E // 8, 128), f32),
        )(e3p, hs_p, Wk3, bk3_2d, r2, smat)
        aggp = sc_scatter(msg_p.reshape(E, WIDTH), dst3, zeros_agg)
        if k != DEPTH - 1:
            h = pl.pallas_call(
                functools.partial(_update_body, relu=True),
                out_shape=jax.ShapeDtypeStruct((N, WIDTH), f32),
            )(aggp[0], aggp[1], inv16, h, root, cb_2d)
        else:
            out = pl.pallas_call(
                _final_body,
                out_shape=jax.ShapeDtypeStruct((N, 1), f32),
            )(aggp[0], aggp[1], inv16, h, root, cb_2d, Wfc2, bfc2_2d)
    return out


# speedup vs baseline: 1.6801x; 1.0644x over previous
import functools

import jax
import jax.numpy as jnp
import numpy as np
from jax import lax
from jax.experimental import pallas as pl
from jax.experimental.pallas import tpu as pltpu
from jax.experimental.pallas import tpu_sc as plsc

N = 10000
E = 160000
WIDTH = 16
DEPTH = 4

NW = 32
EPW = E // NW
CH = 125
NCH = EPW // CH
AGG_N = 10240
RPS = AGG_N // 16


@functools.cache
def _sc_kernels():
    mesh = plsc.VectorSubcoreMesh(core_axis_name="c", subcore_axis_name="s")
    cparams = pltpu.CompilerParams(use_tc_tiling_on_sc=False)

    @functools.partial(
        pl.kernel,
        out_type=jax.ShapeDtypeStruct((E, WIDTH), jnp.float32),
        mesh=mesh,
        compiler_params=cparams,
        scratch_types=[
            pltpu.VMEM((NCH, CH), jnp.int32),
            pltpu.VMEM((EPW, WIDTH), jnp.float32),
            pltpu.VMEM_SHARED((AGG_N, WIDTH), jnp.float32),
            pltpu.SemaphoreType.DMA,
        ],
    )
    def sc_gather(h_hbm, src_hbm, out_hbm, idx_v, rows_v, h_sh, sem):
        c = lax.axis_index("c")
        s = lax.axis_index("s")
        wid = s * 2 + c

        @pl.when(s == 0)
        def _():
            pltpu.sync_copy(h_hbm, h_sh)

        pltpu.sync_copy(src_hbm.at[wid], idx_v)
        plsc.subcore_barrier()

        def fire(j, carry):
            pltpu.async_copy(h_sh.at[idx_v.at[j]],
                             rows_v.at[pl.ds(j * CH, CH)], sem)
            return carry

        lax.fori_loop(0, NCH, fire, 0)
        pltpu.make_async_copy(out_hbm.at[pl.ds(wid * EPW, EPW)], rows_v,
                              sem).wait()
        pltpu.sync_copy(rows_v, out_hbm.at[pl.ds(wid * EPW, EPW)])

    @functools.partial(
        pl.kernel,
        out_type=jax.ShapeDtypeStruct((2, AGG_N, WIDTH), jnp.float32),
        mesh=mesh,
        compiler_params=cparams,
        scratch_types=[
            pltpu.VMEM((NCH, CH), jnp.int32),
            pltpu.VMEM((EPW, WIDTH), jnp.float32),
            pltpu.VMEM_SHARED((AGG_N, WIDTH), jnp.float32),
        ],
    )
    def sc_scatter(msg_hbm, dst_hbm, zeros_hbm, out_hbm, idx_v, msg_v, agg_sh):
        c = lax.axis_index("c")
        s = lax.axis_index("s")
        wid = s * 2 + c

        @pl.when(s == 0)
        def _():
            pltpu.sync_copy(zeros_hbm, agg_sh)

        pltpu.sync_copy(dst_hbm.at[wid], idx_v)
        pltpu.sync_copy(msg_hbm.at[pl.ds(wid * EPW, EPW)], msg_v)
        plsc.subcore_barrier()

        def body(j, carry):
            pltpu.sync_copy(msg_v.at[pl.ds(j * CH, CH)],
                            agg_sh.at[idx_v.at[j]], add=True)
            return carry

        lax.fori_loop(0, NCH, body, 0)
        plsc.subcore_barrier()
        pltpu.sync_copy(agg_sh.at[pl.ds(s * RPS, RPS)],
                        out_hbm.at[c, pl.ds(s * RPS, RPS)])

    @functools.partial(
        pl.kernel,
        out_type=jax.ShapeDtypeStruct((2, AGG_N, WIDTH), jnp.float32),
        mesh=mesh,
        compiler_params=cparams,
        scratch_types=[
            pltpu.VMEM((NCH, CH), jnp.int32),
            pltpu.VMEM((CH, WIDTH), jnp.float32),
            pltpu.VMEM_SHARED((AGG_N, WIDTH), jnp.float32),
        ],
    )
    def sc_count(dst_hbm, zeros_hbm, ones_hbm, out_hbm, idx_v, ones_v, cnt_sh):
        c = lax.axis_index("c")
        s = lax.axis_index("s")
        wid = s * 2 + c

        @pl.when(s == 0)
        def _():
            pltpu.sync_copy(zeros_hbm, cnt_sh)

        pltpu.sync_copy(dst_hbm.at[wid], idx_v)
        pltpu.sync_copy(ones_hbm, ones_v)
        plsc.subcore_barrier()

        def body(j, carry):
            pltpu.sync_copy(ones_v, cnt_sh.at[idx_v.at[j]], add=True)
            return carry

        lax.fori_loop(0, NCH, body, 0)
        plsc.subcore_barrier()
        pltpu.sync_copy(cnt_sh.at[pl.ds(s * RPS, RPS)],
                        out_hbm.at[c, pl.ds(s * RPS, RPS)])

    return sc_gather, sc_scatter, sc_count


def _dot(a, b):
    return jax.lax.dot_general(a, b, (((1,), (0,)), ((), ())),
                               preferred_element_type=jnp.float32)


def _lift_body(x_ref, wf1, bf1, wf2, bf2, wc1, bc1, o_ref):
    x = x_ref[...]
    h = jnp.sin(_dot(x, wf1[...]) + bf1[...])
    o_ref[...] = _dot(h, wf2[...]) + bf2[...] + _dot(x, wc1[...]) + bc1[...]


def _bdot(a, b, out=jnp.float32):
    return jax.lax.dot_general(a.astype(jnp.bfloat16), b.astype(jnp.bfloat16),
                               (((1,), (0,)), ((), ())),
                               preferred_element_type=out)


def _edge_mlp_body(ea_ref, w1, b1, w2, b2, o_ref):
    e = jnp.maximum(_dot(ea_ref[...], w1[...]) + b1[...], 0.0)
    e = jnp.maximum(_dot(e, w2[...]) + b2[...], 0.0)
    o_ref[...] = e.astype(jnp.bfloat16)


def _msg_body(e3_ref, hsp_ref, wk3, bk3, r2, s, o_ref):
    w = _bdot(e3_ref[...], wk3[...]) + bk3[...]
    hsp = hsp_ref[...]
    br = hsp.shape[0]
    cols = []
    for a in range(8):
        hsx_a = _bdot(hsp[:, 16 * a:16 * (a + 1)], r2[...])
        t_a = w[a * br:(a + 1) * br, :] * hsx_a
        cols.append(_bdot(t_a, s[...]))
    o_ref[...] = jnp.concatenate(cols, axis=1)


def _inv_body(c0_ref, c1_ref, o_ref):
    o_ref[...] = 1.0 / jnp.maximum(c0_ref[...] + c1_ref[...], 1.0)


def _update_body(a0_ref, a1_ref, inv_ref, h_ref, rootbd, cb8, o_ref, *, relu):
    agg = (a0_ref[...] + a1_ref[...]) * inv_ref[...]
    h = agg + _dot(h_ref[...], rootbd[...]) + cb8[...]
    if relu:
        h = jnp.maximum(h, 0.0)
    o_ref[...] = h


def _final_body(a0_ref, a1_ref, inv_ref, h_ref, rootbd, cb8, wfc2bd, bfc2,
                o_ref):
    agg = (a0_ref[...] + a1_ref[...]) * inv_ref[...]
    h = agg + _dot(h_ref[...], rootbd[...]) + cb8[...]
    o_ref[...] = _dot(h, wfc2bd[...]) + bfc2[...]


def _full(x):
    return pl.BlockSpec(x.shape, lambda *_: tuple(0 for _ in x.shape))


def _expansion_mats():
    r2 = np.zeros((WIDTH, WIDTH * WIDTH), np.float32)
    s = np.zeros((WIDTH * WIDTH, WIDTH), np.float32)
    for i in range(WIDTH):
        for o in range(WIDTH):
            r2[i, WIDTH * i + o] = 1.0
            s[WIDTH * i + o, o] = 1.0
    return jnp.asarray(r2), jnp.asarray(s)


def kernel(x, edge_index, edge_attr, Wff1, bff1, Wff2, bff2, Wfc1, bfc1,
           Wk1, bk1, Wk2, bk2, Wk3, bk3, root, conv_bias, Wfc2, bfc2):
    f32 = jnp.float32
    sc_gather, sc_scatter, sc_count = _sc_kernels()

    src3 = edge_index[0].astype(jnp.int32).reshape(NW, NCH, CH)
    dst3 = edge_index[1].astype(jnp.int32).reshape(NW, NCH, CH)
    zeros_agg = jnp.zeros((AGG_N, WIDTH), f32)
    NPK = AGG_N // 8
    rootbd = jnp.kron(jnp.eye(8, dtype=f32), root)
    cb8 = jnp.tile(conv_bias, 8).reshape(1, 128)
    wfc2bd = jnp.kron(jnp.eye(8, dtype=f32), Wfc2)
    ones_ch = jnp.ones((CH, WIDTH), f32)
    r2, smat = _expansion_mats()

    cntp = sc_count(dst3, zeros_agg, ones_ch).reshape(2, NPK, 128)
    inv_p = pl.pallas_call(
        _inv_body,
        out_shape=jax.ShapeDtypeStruct((NPK, 128), f32),
    )(cntp[0], cntp[1])

    BN = 2000
    h = pl.pallas_call(
        _lift_body,
        grid=(N // BN,),
        in_specs=[pl.BlockSpec((BN, x.shape[1]), lambda i: (i, 0)),
                  _full(Wff1), _full(bff1.reshape(1, -1)),
                  _full(Wff2), _full(bff2.reshape(1, -1)),
                  _full(Wfc1), _full(bfc1.reshape(1, -1))],
        out_specs=pl.BlockSpec((BN, WIDTH), lambda i: (i, 0)),
        out_shape=jax.ShapeDtypeStruct((N, WIDTH), f32),
    )(x, Wff1, bff1.reshape(1, -1), Wff2, bff2.reshape(1, -1),
      Wfc1, bfc1.reshape(1, -1))
    h_p = jnp.zeros((AGG_N, WIDTH), f32).at[:N].set(h).reshape(NPK, 128)

    b1_2d = bk1.reshape(1, -1)
    b2_2d = bk2.reshape(1, -1)
    bk3_2d = bk3.reshape(1, -1)
    cb_2d = conv_bias.reshape(1, -1)
    bfc2_2d = bfc2.reshape(1, -1)

    BE = 3200
    BR = BE // 8
    perm = np.arange(E).reshape(E // BE, BR, 8).transpose(0, 2, 1).reshape(E)
    ea_perm = edge_attr[jnp.asarray(perm)]
    KDIM = Wk2.shape[1]
    e3p = pl.pallas_call(
        _edge_mlp_body,
        grid=(E // BE,),
        in_specs=[pl.BlockSpec((BE, ea_perm.shape[1]), lambda i: (i, 0)),
                  _full(Wk1), _full(b1_2d), _full(Wk2), _full(b2_2d)],
        out_specs=pl.BlockSpec((BE, KDIM), lambda i: (i, 0)),
        out_shape=jax.ShapeDtypeStruct((E, KDIM), jnp.bfloat16),
    )(ea_perm, Wk1, b1_2d, Wk2, b2_2d)
    for k in range(DEPTH):
        hs = sc_gather(h_p.reshape(AGG_N, WIDTH), src3)
        hs_p = hs.reshape(E // 8, 128)
        msg_p = pl.pallas_call(
            _msg_body,
            grid=(E // BE,),
            in_specs=[pl.BlockSpec((BE, KDIM), lambda i: (i, 0)),
                      pl.BlockSpec((BR, 128), lambda i: (i, 0)),
                      _full(Wk3), _full(bk3_2d), _full(r2), _full(smat)],
            out_specs=pl.BlockSpec((BR, 128), lambda i: (i, 0)),
            out_shape=jax.ShapeDtypeStruct((E // 8, 128), f32),
        )(e3p, hs_p, Wk3, bk3_2d, r2, smat)
        aggp = sc_scatter(msg_p.reshape(E, WIDTH), dst3,
                          zeros_agg).reshape(2, NPK, 128)
        if k != DEPTH - 1:
            h_p = pl.pallas_call(
                functools.partial(_update_body, relu=True),
                out_shape=jax.ShapeDtypeStruct((NPK, 128), f32),
            )(aggp[0], aggp[1], inv_p, h_p, rootbd, cb8)
        else:
            out_p = pl.pallas_call(
                _final_body,
                out_shape=jax.ShapeDtypeStruct((NPK, 8), f32),
            )(aggp[0], aggp[1], inv_p, h_p, rootbd, cb8, wfc2bd, bfc2_2d)
    return out_p.reshape(AGG_N, 1)[:N]
